# baseline (speedup 1.0000x reference)
import numpy as np
import jax
import jax.numpy as jnp
from jax.experimental import pallas as pl

_RATIOS = np.array([0.5, 1.0, 2.0])
_SCALES = np.array([1.0, 2.0 ** (1.0 / 3.0), 2.0 ** (2.0 / 3.0)])
_LEVELS = [(64, 64, 8, 32), (32, 32, 16, 64), (16, 16, 32, 128), (8, 8, 64, 256)]
_N_ROWS = sum(h * w * 9 for h, w, _, _ in _LEVELS)


def _box_sizes(box_size):
    anchors = box_size * np.tile(_SCALES, (2, len(_RATIOS))).T
    areas = anchors[:, 0] * anchors[:, 1]
    anchors[:, 0] = np.sqrt(areas * np.repeat(_RATIOS, len(_SCALES)))
    anchors[:, 1] = anchors[:, 0] / np.repeat(_RATIOS, len(_SCALES))
    return anchors.astype(np.float32)


def _lane_tables(size):
    wh = _box_sizes(size)
    kwh = np.zeros((1, 36), np.float32)
    kofs = np.zeros((1, 36), np.float32)
    for a in range(9):
        w, h = wh[a]
        kwh[0, a * 4 + 2] = w
        kwh[0, a * 4 + 3] = h
        kofs[0, a * 4 + 0] = -w / 2.0
        kofs[0, a * 4 + 1] = -h / 2.0
        kofs[0, a * 4 + 2] = w / 2.0
        kofs[0, a * 4 + 3] = h / 2.0
    return kwh, kofs


def _anchor_kernel(tab_ref, out_xywh, out_xyxy):
    cell_off = 0
    for lvl, (H, W, stride, size) in enumerate(_LEVELS):
        hw = H * W
        kwh = tab_ref[2 * lvl:2 * lvl + 1, :]
        kofs = tab_ref[2 * lvl + 1:2 * lvl + 2, :]
        cell = jax.lax.broadcasted_iota(jnp.int32, (hw, 1), 0)
        wi = jnp.bitwise_and(cell, W - 1)
        hi = jax.lax.shift_right_logical(cell, int(np.log2(W)))
        x = (wi.astype(jnp.float32) + 0.5) * float(stride)
        y = (hi.astype(jnp.float32) + 0.5) * float(stride)
        lane = jax.lax.broadcasted_iota(jnp.int32, (hw, 36), 1)
        c = jnp.bitwise_and(lane, 3)
        vals = jnp.where(c == 0, x, jnp.where(c == 1, y, kwh))
        out_xywh[cell_off:cell_off + hw, :] = vals
        xy = jnp.where(jnp.bitwise_and(lane, 1) == 0, x, y)
        out_xyxy[cell_off:cell_off + hw, :] = xy + kofs
        cell_off += hw


def _tables_all():
    rows = []
    for (_, _, _, size) in _LEVELS:
        kwh, kofs = _lane_tables(size)
        rows.append(kwh[0])
        rows.append(kofs[0])
    return np.stack(rows)


_TABLES = _tables_all()


def kernel(feat_p3, feat_p4, feat_p5, feat_p6):
    del feat_p3, feat_p4, feat_p5, feat_p6
    n_cells = _N_ROWS // 9
    out_shape = (
        jax.ShapeDtypeStruct((n_cells, 36), jnp.float32),
        jax.ShapeDtypeStruct((n_cells, 36), jnp.float32),
    )
    anchors, anchors_xyxy = pl.pallas_call(
        _anchor_kernel,
        out_shape=out_shape,
    )(jnp.asarray(_TABLES))
    return anchors.reshape(_N_ROWS, 4), anchors_xyxy.reshape(_N_ROWS, 4)


# speedup vs baseline: 1.2747x; 1.2747x over previous
import numpy as np
import jax
import jax.numpy as jnp
from jax.experimental import pallas as pl

_RATIOS = np.array([0.5, 1.0, 2.0])
_SCALES = np.array([1.0, 2.0 ** (1.0 / 3.0), 2.0 ** (2.0 / 3.0)])
_LEVELS = [(64, 64, 8, 32), (32, 32, 16, 64), (16, 16, 32, 128), (8, 8, 64, 256)]
_N_ROWS = sum(h * w * 9 for h, w, _, _ in _LEVELS)
_N_PAD = 49152
_ROW_OFF = [0, 36864, 46080, 48384]


def _box_sizes(box_size):
    anchors = box_size * np.tile(_SCALES, (2, len(_RATIOS))).T
    areas = anchors[:, 0] * anchors[:, 1]
    anchors[:, 0] = np.sqrt(areas * np.repeat(_RATIOS, len(_SCALES)))
    anchors[:, 1] = anchors[:, 0] / np.repeat(_RATIOS, len(_SCALES))
    return anchors.astype(np.float32)


def _sel_by_level(n, vals, dtype):
    out = jnp.full(n.shape, vals[3], dtype)
    for lvl in (2, 1, 0):
        out = jnp.where(n < _ROW_OFF[lvl + 1], jnp.asarray(vals[lvl], dtype), out)
    return out


_BLK = 8192


def _anchor_kernel(out_xywh, out_xyxy):
    pid = pl.program_id(0)
    n = pid * _BLK + jax.lax.broadcasted_iota(jnp.int32, (1, _BLK), 1)
    off = _sel_by_level(n, _ROW_OFF, jnp.int32)
    stride_f = _sel_by_level(n, [float(s) for (_, _, s, _) in _LEVELS], jnp.float32)
    mask_w = _sel_by_level(n, [w - 1 for (_, w, _, _) in _LEVELS], jnp.int32)
    lg_w = _sel_by_level(n, [int(np.log2(w)) for (_, w, _, _) in _LEVELS], jnp.int32)
    size_f = _sel_by_level(n, [float(s) for (_, _, _, s) in _LEVELS], jnp.float32)

    q = n - off
    cell = jnp.floor((q.astype(jnp.float32) + 0.5) * (1.0 / 9.0)).astype(jnp.int32)
    a = q - 9 * cell
    wi = jnp.bitwise_and(cell, mask_w)
    hi = jax.lax.shift_right_logical(cell, lg_w)
    x = (wi.astype(jnp.float32) + 0.5) * stride_f
    y = (hi.astype(jnp.float32) + 0.5) * stride_f

    base = _box_sizes(1.0)
    w = jnp.full(n.shape, float(base[8, 0]), jnp.float32)
    h = jnp.full(n.shape, float(base[8, 1]), jnp.float32)
    for k in range(7, -1, -1):
        sel = a <= k
        w = jnp.where(sel, float(base[k, 0]), w)
        h = jnp.where(sel, float(base[k, 1]), h)
    w = w * size_f
    h = h * size_f

    big = jnp.concatenate(
        [x, y, w, h, x - 0.5 * w, y - 0.5 * h, x + 0.5 * w, y + 0.5 * h], axis=0
    )
    t = jnp.swapaxes(big, 0, 1)
    out_xywh[:, :] = t[:, 0:4]
    out_xyxy[:, :] = t[:, 4:8]


def kernel(feat_p3, feat_p4, feat_p5, feat_p6):
    del feat_p3, feat_p4, feat_p5, feat_p6
    out_shape = (
        jax.ShapeDtypeStruct((_N_ROWS, 4), jnp.float32),
        jax.ShapeDtypeStruct((_N_ROWS, 4), jnp.float32),
    )
    anchors, anchors_xyxy = pl.pallas_call(
        _anchor_kernel,
        grid=(_N_PAD // _BLK,),
        out_specs=(
            pl.BlockSpec((_BLK, 4), lambda i: (i, 0)),
            pl.BlockSpec((_BLK, 4), lambda i: (i, 0)),
        ),
        out_shape=out_shape,
    )()
    return anchors, anchors_xyxy


# speedup vs baseline: 7.6662x; 6.0141x over previous
import numpy as np
import jax
import jax.numpy as jnp
from jax.experimental import pallas as pl

_RATIOS = np.array([0.5, 1.0, 2.0])
_SCALES = np.array([1.0, 2.0 ** (1.0 / 3.0), 2.0 ** (2.0 / 3.0)])
_LEVELS = [(64, 64, 8, 32), (32, 32, 16, 64), (16, 16, 32, 128), (8, 8, 64, 256)]
_N_ROWS = sum(h * w * 9 for h, w, _, _ in _LEVELS)
_N_PAD = 49152
_ROW_OFF = [0, 36864, 46080, 48384]


def _box_sizes(box_size):
    anchors = box_size * np.tile(_SCALES, (2, len(_RATIOS))).T
    areas = anchors[:, 0] * anchors[:, 1]
    anchors[:, 0] = np.sqrt(areas * np.repeat(_RATIOS, len(_SCALES)))
    anchors[:, 1] = anchors[:, 0] / np.repeat(_RATIOS, len(_SCALES))
    return anchors.astype(np.float32)


def _sel_by_level(n, vals, dtype):
    out = jnp.full(n.shape, vals[3], dtype)
    for lvl in (2, 1, 0):
        out = jnp.where(n < _ROW_OFF[lvl + 1], jnp.asarray(vals[lvl], dtype), out)
    return out


def _anchor_kernel(out_cols):
    n = jax.lax.broadcasted_iota(jnp.int32, (1, _N_PAD), 1)
    off = _sel_by_level(n, _ROW_OFF, jnp.int32)
    stride_f = _sel_by_level(n, [float(s) for (_, _, s, _) in _LEVELS], jnp.float32)
    mask_w = _sel_by_level(n, [w - 1 for (_, w, _, _) in _LEVELS], jnp.int32)
    lg_w = _sel_by_level(n, [int(np.log2(w)) for (_, w, _, _) in _LEVELS], jnp.int32)
    size_f = _sel_by_level(n, [float(s) for (_, _, _, s) in _LEVELS], jnp.float32)

    q = n - off
    cell = jnp.floor((q.astype(jnp.float32) + 0.5) * (1.0 / 9.0)).astype(jnp.int32)
    a = q - 9 * cell
    wi = jnp.bitwise_and(cell, mask_w)
    hi = jax.lax.shift_right_logical(cell, lg_w)
    x = (wi.astype(jnp.float32) + 0.5) * stride_f
    y = (hi.astype(jnp.float32) + 0.5) * stride_f

    base = _box_sizes(1.0)
    w = jnp.full(n.shape, float(base[8, 0]), jnp.float32)
    h = jnp.full(n.shape, float(base[8, 1]), jnp.float32)
    for k in range(7, -1, -1):
        sel = a <= k
        w = jnp.where(sel, float(base[k, 0]), w)
        h = jnp.where(sel, float(base[k, 1]), h)
    w = w * size_f
    h = h * size_f

    out_cols[:, :] = jnp.concatenate(
        [x, y, w, h, x - 0.5 * w, y - 0.5 * h, x + 0.5 * w, y + 0.5 * h], axis=0
    )


def kernel(feat_p3, feat_p4, feat_p5, feat_p6):
    del feat_p3, feat_p4, feat_p5, feat_p6
    big = pl.pallas_call(
        _anchor_kernel,
        out_shape=jax.ShapeDtypeStruct((8, _N_PAD), jnp.float32),
    )()
    anchors = big[0:4, :_N_ROWS].T
    anchors_xyxy = big[4:8, :_N_ROWS].T
    return anchors, anchors_xyxy
